# baseline (speedup 1.0000x reference)
import jax
import jax.numpy as jnp
from jax import lax
from jax.experimental import pallas as pl
from jax.experimental.pallas import tpu as pltpu
from jax.experimental.pallas import tpu_sc as plsc

NC = 2
NS = 16
L = 16
NW = NC * NS
BATCH = 16384
D = 100
BPW = BATCH // NW
CH = 128
NCHK = BPW // CH
CGROUPS = CH // L

TBLK = 2048


def _transpose_block(in_ref, out_ref):
    out_ref[...] = in_ref[...].T


def _to_row_major(table_t):
    V = table_t.shape[1]
    nblk = (V + TBLK - 1) // TBLK
    return pl.pallas_call(
        _transpose_block,
        grid=(nblk,),
        in_specs=[pl.BlockSpec((D, TBLK), lambda b: (0, b))],
        out_specs=pl.BlockSpec((TBLK, D), lambda b: (b, 0)),
        out_shape=jax.ShapeDtypeStruct((V, D), jnp.float32),
    )(table_t)


def _mf_loss_body(u_hbm, i_hbm, r_hbm, w_hbm, h_hbm, out_hbm,
                  u_v, i_v, rv, wrows, hrows, acc_v, sem0, sem1):
    c = lax.axis_index("c")
    s = lax.axis_index("s")
    wid = s * NC + c
    sems = (sem0, sem1)

    pltpu.sync_copy(u_hbm.at[wid], u_v)
    pltpu.sync_copy(i_hbm.at[wid], i_v)
    pltpu.sync_copy(r_hbm.at[wid], rv)

    def fetch(chunk, b):
        base = chunk * CH

        def issue(g, carry):
            uvec = u_v[pl.ds(base + g * L, L)]
            ivec = i_v[pl.ds(base + g * L, L)]
            for j in range(L):
                row = g * L + j
                pltpu.async_copy(w_hbm.at[uvec[j]], wrows.at[b, row], sems[b])
                pltpu.async_copy(h_hbm.at[ivec[j]], hrows.at[b, row], sems[b])
            return carry
        lax.fori_loop(0, CGROUPS, issue, 0)

    def wait_chunk(b):
        def drain(k, carry):
            pltpu.make_async_copy(w_hbm.at[0], wrows.at[b, k], sems[b]).wait()
            pltpu.make_async_copy(h_hbm.at[0], hrows.at[b, k], sems[b]).wait()
            return carry
        lax.fori_loop(0, CH, drain, 0)

    lane = lax.iota(jnp.int32, L)
    row0 = wid * BPW

    fetch(0, 0)
    fetch(1, 1)
    lacc = jnp.zeros((L,), jnp.float32)
    for chunk in range(NCHK):
        b = chunk % 2
        wait_chunk(b)
        wch = wrows.at[b]
        hch = hrows.at[b]

        def group_body(g, acc, _wch=wch, _hch=hch, _chunk=chunk):
            rows = g * L + lane
            dot = jnp.zeros((L,), jnp.float32)
            for d in range(D):
                dvec = jnp.full((L,), d, jnp.int32)
                wv = plsc.load_gather(_wch, [rows, dvec])
                hv = plsc.load_gather(_hch, [rows, dvec])
                dot = dot + wv * hv
            rvals = rv[pl.ds(_chunk * CH + g * L, L)]
            err = dot - rvals
            grow = row0 + _chunk * CH + g * L + lane
            sq = jnp.where(grow == 0, jnp.float32(0), err * err)
            return acc + sq

        lacc = lax.fori_loop(0, CGROUPS, group_body, lacc)
        if chunk + 2 < NCHK:
            fetch(chunk + 2, b)

    acc_v[...] = lacc
    pltpu.sync_copy(acc_v, out_hbm.at[wid])


def kernel(u, i, r, W, H, U):
    del U
    u32 = u.astype(jnp.int32).reshape(NW, BPW)
    i32 = i.astype(jnp.int32).reshape(NW, BPW)
    r2 = r.astype(jnp.float32).reshape(NW, BPW)
    w_rm = _to_row_major(W.T)
    h_rm = _to_row_major(H.T)
    mesh = plsc.VectorSubcoreMesh(core_axis_name="c", subcore_axis_name="s")
    partials = pl.kernel(
        _mf_loss_body,
        out_type=jax.ShapeDtypeStruct((NW, L), jnp.float32),
        mesh=mesh,
        compiler_params=pltpu.CompilerParams(needs_layout_passes=False),
        scratch_types=[
            pltpu.VMEM((BPW,), jnp.int32),
            pltpu.VMEM((BPW,), jnp.int32),
            pltpu.VMEM((BPW,), jnp.float32),
            pltpu.VMEM((2, CH, D), jnp.float32),
            pltpu.VMEM((2, CH, D), jnp.float32),
            pltpu.VMEM((L,), jnp.float32),
            pltpu.SemaphoreType.DMA,
            pltpu.SemaphoreType.DMA,
        ],
    )(u32, i32, r2, w_rm, h_rm)
    return jnp.sum(partials) / BATCH


# speedup vs baseline: 1.5689x; 1.5689x over previous
import jax
import jax.numpy as jnp
from jax import lax
from jax.experimental import pallas as pl
from jax.experimental.pallas import tpu as pltpu
from jax.experimental.pallas import tpu_sc as plsc

NC = 2
NS = 16
L = 16
NW = NC * NS
BATCH = 16384
D = 100
BPW = BATCH // NW
CH = 128
NCHK = BPW // CH
CGROUPS = CH // L

TBLK = 8192


def _transpose_block(in_ref, out_ref):
    out_ref[...] = in_ref[...].T


def _to_row_major(table_t):
    V = table_t.shape[1]
    nblk = (V + TBLK - 1) // TBLK
    return pl.pallas_call(
        _transpose_block,
        grid=(nblk,),
        in_specs=[pl.BlockSpec((D, TBLK), lambda b: (0, b))],
        out_specs=pl.BlockSpec((TBLK, D), lambda b: (b, 0)),
        out_shape=jax.ShapeDtypeStruct((V, D), jnp.float32),
    )(table_t)


def _mf_loss_body(u_hbm, i_hbm, r_hbm, w_hbm, h_hbm, out_hbm,
                  u_v, i_v, rv, wrows, hrows, acc_v, sem0, sem1):
    c = lax.axis_index("c")
    s = lax.axis_index("s")
    wid = s * NC + c
    sems = (sem0, sem1)

    pltpu.sync_copy(u_hbm.at[wid], u_v)
    pltpu.sync_copy(i_hbm.at[wid], i_v)
    pltpu.sync_copy(r_hbm.at[wid], rv)

    def fetch(chunk, b):
        base = chunk * CH

        def issue(g, carry):
            uvec = u_v[pl.ds(base + g * L, L)]
            ivec = i_v[pl.ds(base + g * L, L)]
            for j in range(L):
                row = g * L + j
                pltpu.async_copy(w_hbm.at[uvec[j]], wrows.at[b, row], sems[b])
                pltpu.async_copy(h_hbm.at[ivec[j]], hrows.at[b, row], sems[b])
            return carry
        lax.fori_loop(0, CGROUPS, issue, 0)

    def wait_chunk(b):
        def drain(k, carry):
            pltpu.make_async_copy(w_hbm.at[0], wrows.at[b, k], sems[b]).wait()
            pltpu.make_async_copy(h_hbm.at[0], hrows.at[b, k], sems[b]).wait()
            return carry
        lax.fori_loop(0, CH, drain, 0)

    lane = lax.iota(jnp.int32, L)
    row0 = wid * BPW

    fetch(0, 0)
    fetch(1, 1)
    lacc = jnp.zeros((L,), jnp.float32)
    for chunk in range(NCHK):
        b = chunk % 2
        wait_chunk(b)
        wch = wrows.at[b]
        hch = hrows.at[b]

        def group_body(g, acc, _wch=wch, _hch=hch, _chunk=chunk):
            rows = g * L + lane
            dot = jnp.zeros((L,), jnp.float32)
            for d in range(D):
                dvec = jnp.full((L,), d, jnp.int32)
                wv = plsc.load_gather(_wch, [rows, dvec])
                hv = plsc.load_gather(_hch, [rows, dvec])
                dot = dot + wv * hv
            rvals = rv[pl.ds(_chunk * CH + g * L, L)]
            err = dot - rvals
            grow = row0 + _chunk * CH + g * L + lane
            sq = jnp.where(grow == 0, jnp.float32(0), err * err)
            return acc + sq

        lacc = lax.fori_loop(0, CGROUPS, group_body, lacc)
        if chunk + 2 < NCHK:
            fetch(chunk + 2, b)

    acc_v[...] = lacc
    pltpu.sync_copy(acc_v, out_hbm.at[wid])


def kernel(u, i, r, W, H, U):
    del U
    u32 = u.astype(jnp.int32).reshape(NW, BPW)
    i32 = i.astype(jnp.int32).reshape(NW, BPW)
    r2 = r.astype(jnp.float32).reshape(NW, BPW)
    w_rm = _to_row_major(W.T)
    h_rm = _to_row_major(H.T)
    mesh = plsc.VectorSubcoreMesh(core_axis_name="c", subcore_axis_name="s")
    partials = pl.kernel(
        _mf_loss_body,
        out_type=jax.ShapeDtypeStruct((NW, L), jnp.float32),
        mesh=mesh,
        compiler_params=pltpu.CompilerParams(needs_layout_passes=False),
        scratch_types=[
            pltpu.VMEM((BPW,), jnp.int32),
            pltpu.VMEM((BPW,), jnp.int32),
            pltpu.VMEM((BPW,), jnp.float32),
            pltpu.VMEM((2, CH, D), jnp.float32),
            pltpu.VMEM((2, CH, D), jnp.float32),
            pltpu.VMEM((L,), jnp.float32),
            pltpu.SemaphoreType.DMA,
            pltpu.SemaphoreType.DMA,
        ],
    )(u32, i32, r2, w_rm, h_rm)
    return jnp.sum(partials) / BATCH


# speedup vs baseline: 1.6225x; 1.0341x over previous
import jax
import jax.numpy as jnp
from jax import lax
from jax.experimental import pallas as pl
from jax.experimental.pallas import tpu as pltpu
from jax.experimental.pallas import tpu_sc as plsc

NC = 2
NS = 16
L = 16
NW = NC * NS
BATCH = 16384
D = 100
BPW = BATCH // NW
CH = 128
NCHK = BPW // CH
CGROUPS = CH // L

TBLK = 16384


def _transpose_block(in_ref, out_ref):
    out_ref[...] = in_ref[...].T


def _to_row_major(table_t):
    V = table_t.shape[1]
    nblk = (V + TBLK - 1) // TBLK
    return pl.pallas_call(
        _transpose_block,
        grid=(nblk,),
        in_specs=[pl.BlockSpec((D, TBLK), lambda b: (0, b))],
        out_specs=pl.BlockSpec((TBLK, D), lambda b: (b, 0)),
        out_shape=jax.ShapeDtypeStruct((V, D), jnp.float32),
    )(table_t)


def _mf_loss_body(u_hbm, i_hbm, r_hbm, w_hbm, h_hbm, out_hbm,
                  u_v, i_v, rv, wrows, hrows, acc_v, sem0, sem1):
    c = lax.axis_index("c")
    s = lax.axis_index("s")
    wid = s * NC + c
    sems = (sem0, sem1)

    pltpu.sync_copy(u_hbm.at[wid], u_v)
    pltpu.sync_copy(i_hbm.at[wid], i_v)
    pltpu.sync_copy(r_hbm.at[wid], rv)

    def fetch(chunk, b):
        base = chunk * CH

        def issue(g, carry):
            uvec = u_v[pl.ds(base + g * L, L)]
            ivec = i_v[pl.ds(base + g * L, L)]
            for j in range(L):
                row = g * L + j
                pltpu.async_copy(w_hbm.at[uvec[j]], wrows.at[b, row], sems[b])
                pltpu.async_copy(h_hbm.at[ivec[j]], hrows.at[b, row], sems[b])
            return carry
        lax.fori_loop(0, CGROUPS, issue, 0)

    def wait_chunk(b):
        def drain(k, carry):
            pltpu.make_async_copy(w_hbm.at[0], wrows.at[b, k], sems[b]).wait()
            pltpu.make_async_copy(h_hbm.at[0], hrows.at[b, k], sems[b]).wait()
            return carry
        lax.fori_loop(0, CH, drain, 0)

    lane = lax.iota(jnp.int32, L)
    row0 = wid * BPW

    fetch(0, 0)
    fetch(1, 1)
    lacc = jnp.zeros((L,), jnp.float32)
    for chunk in range(NCHK):
        b = chunk % 2
        wait_chunk(b)
        wch = wrows.at[b]
        hch = hrows.at[b]

        def group_body(g, acc, _wch=wch, _hch=hch, _chunk=chunk):
            rows = g * L + lane
            dot = jnp.zeros((L,), jnp.float32)
            for d in range(D):
                dvec = jnp.full((L,), d, jnp.int32)
                wv = plsc.load_gather(_wch, [rows, dvec])
                hv = plsc.load_gather(_hch, [rows, dvec])
                dot = dot + wv * hv
            rvals = rv[pl.ds(_chunk * CH + g * L, L)]
            err = dot - rvals
            grow = row0 + _chunk * CH + g * L + lane
            sq = jnp.where(grow == 0, jnp.float32(0), err * err)
            return acc + sq

        lacc = lax.fori_loop(0, CGROUPS, group_body, lacc)
        if chunk + 2 < NCHK:
            fetch(chunk + 2, b)

    acc_v[...] = lacc
    pltpu.sync_copy(acc_v, out_hbm.at[wid])


def kernel(u, i, r, W, H, U):
    del U
    u32 = u.astype(jnp.int32).reshape(NW, BPW)
    i32 = i.astype(jnp.int32).reshape(NW, BPW)
    r2 = r.astype(jnp.float32).reshape(NW, BPW)
    w_rm = _to_row_major(W.T)
    h_rm = _to_row_major(H.T)
    mesh = plsc.VectorSubcoreMesh(core_axis_name="c", subcore_axis_name="s")
    partials = pl.kernel(
        _mf_loss_body,
        out_type=jax.ShapeDtypeStruct((NW, L), jnp.float32),
        mesh=mesh,
        compiler_params=pltpu.CompilerParams(needs_layout_passes=False),
        scratch_types=[
            pltpu.VMEM((BPW,), jnp.int32),
            pltpu.VMEM((BPW,), jnp.int32),
            pltpu.VMEM((BPW,), jnp.float32),
            pltpu.VMEM((2, CH, D), jnp.float32),
            pltpu.VMEM((2, CH, D), jnp.float32),
            pltpu.VMEM((L,), jnp.float32),
            pltpu.SemaphoreType.DMA,
            pltpu.SemaphoreType.DMA,
        ],
    )(u32, i32, r2, w_rm, h_rm)
    return jnp.sum(partials) / BATCH


# speedup vs baseline: 1.6428x; 1.0126x over previous
import jax
import jax.numpy as jnp
from jax import lax
from jax.experimental import pallas as pl
from jax.experimental.pallas import tpu as pltpu
from jax.experimental.pallas import tpu_sc as plsc

NC = 2
NS = 16
L = 16
NW = NC * NS
BATCH = 16384
D = 100
BPW = BATCH // NW
CH = 128
NCHK = BPW // CH
CGROUPS = CH // L

TBLK = 32768


def _transpose_block(in_ref, out_ref):
    out_ref[...] = in_ref[...].T


def _to_row_major(table_t):
    V = table_t.shape[1]
    nblk = (V + TBLK - 1) // TBLK
    return pl.pallas_call(
        _transpose_block,
        grid=(nblk,),
        in_specs=[pl.BlockSpec((D, TBLK), lambda b: (0, b))],
        out_specs=pl.BlockSpec((TBLK, D), lambda b: (b, 0)),
        out_shape=jax.ShapeDtypeStruct((V, D), jnp.float32),
        compiler_params=pltpu.CompilerParams(vmem_limit_bytes=120 << 20),
    )(table_t)


def _mf_loss_body(u_hbm, i_hbm, r_hbm, w_hbm, h_hbm, out_hbm,
                  u_v, i_v, rv, wrows, hrows, acc_v, sem0, sem1):
    c = lax.axis_index("c")
    s = lax.axis_index("s")
    wid = s * NC + c
    sems = (sem0, sem1)

    pltpu.sync_copy(u_hbm.at[wid], u_v)
    pltpu.sync_copy(i_hbm.at[wid], i_v)
    pltpu.sync_copy(r_hbm.at[wid], rv)

    def fetch(chunk, b):
        base = chunk * CH

        def issue(g, carry):
            uvec = u_v[pl.ds(base + g * L, L)]
            ivec = i_v[pl.ds(base + g * L, L)]
            for j in range(L):
                row = g * L + j
                pltpu.async_copy(w_hbm.at[uvec[j]], wrows.at[b, row], sems[b])
                pltpu.async_copy(h_hbm.at[ivec[j]], hrows.at[b, row], sems[b])
            return carry
        lax.fori_loop(0, CGROUPS, issue, 0)

    def wait_chunk(b):
        def drain(k, carry):
            pltpu.make_async_copy(w_hbm.at[0], wrows.at[b, k], sems[b]).wait()
            pltpu.make_async_copy(h_hbm.at[0], hrows.at[b, k], sems[b]).wait()
            return carry
        lax.fori_loop(0, CH, drain, 0)

    lane = lax.iota(jnp.int32, L)
    row0 = wid * BPW

    fetch(0, 0)
    fetch(1, 1)
    lacc = jnp.zeros((L,), jnp.float32)
    for chunk in range(NCHK):
        b = chunk % 2
        wait_chunk(b)
        wch = wrows.at[b]
        hch = hrows.at[b]

        def group_body(g, acc, _wch=wch, _hch=hch, _chunk=chunk):
            rows = g * L + lane
            dot = jnp.zeros((L,), jnp.float32)
            for d in range(D):
                dvec = jnp.full((L,), d, jnp.int32)
                wv = plsc.load_gather(_wch, [rows, dvec])
                hv = plsc.load_gather(_hch, [rows, dvec])
                dot = dot + wv * hv
            rvals = rv[pl.ds(_chunk * CH + g * L, L)]
            err = dot - rvals
            grow = row0 + _chunk * CH + g * L + lane
            sq = jnp.where(grow == 0, jnp.float32(0), err * err)
            return acc + sq

        lacc = lax.fori_loop(0, CGROUPS, group_body, lacc)
        if chunk + 2 < NCHK:
            fetch(chunk + 2, b)

    acc_v[...] = lacc
    pltpu.sync_copy(acc_v, out_hbm.at[wid])


def kernel(u, i, r, W, H, U):
    del U
    u32 = u.astype(jnp.int32).reshape(NW, BPW)
    i32 = i.astype(jnp.int32).reshape(NW, BPW)
    r2 = r.astype(jnp.float32).reshape(NW, BPW)
    w_rm = _to_row_major(W.T)
    h_rm = _to_row_major(H.T)
    mesh = plsc.VectorSubcoreMesh(core_axis_name="c", subcore_axis_name="s")
    partials = pl.kernel(
        _mf_loss_body,
        out_type=jax.ShapeDtypeStruct((NW, L), jnp.float32),
        mesh=mesh,
        compiler_params=pltpu.CompilerParams(needs_layout_passes=False),
        scratch_types=[
            pltpu.VMEM((BPW,), jnp.int32),
            pltpu.VMEM((BPW,), jnp.int32),
            pltpu.VMEM((BPW,), jnp.float32),
            pltpu.VMEM((2, CH, D), jnp.float32),
            pltpu.VMEM((2, CH, D), jnp.float32),
            pltpu.VMEM((L,), jnp.float32),
            pltpu.SemaphoreType.DMA,
            pltpu.SemaphoreType.DMA,
        ],
    )(u32, i32, r2, w_rm, h_rm)
    return jnp.sum(partials) / BATCH


# speedup vs baseline: 1.6543x; 1.0070x over previous
import jax
import jax.numpy as jnp
from jax import lax
from jax.experimental import pallas as pl
from jax.experimental.pallas import tpu as pltpu
from jax.experimental.pallas import tpu_sc as plsc

NC = 2
NS = 16
L = 16
NW = NC * NS
BATCH = 16384
D = 100
DP = 128
BPW = BATCH // NW
CH = 128
NCHK = BPW // CH
CGROUPS = CH // L

TBLK = 32768


def _transpose_block(in_ref, out_ref):
    xt = in_ref[...].T
    out_ref[...] = jnp.concatenate(
        [xt, jnp.zeros((xt.shape[0], DP - D), jnp.float32)], axis=1)


def _to_row_major(table_t):
    V = table_t.shape[1]
    nblk = (V + TBLK - 1) // TBLK
    return pl.pallas_call(
        _transpose_block,
        grid=(nblk,),
        in_specs=[pl.BlockSpec((D, TBLK), lambda b: (0, b))],
        out_specs=pl.BlockSpec((TBLK, DP), lambda b: (b, 0)),
        out_shape=jax.ShapeDtypeStruct((V, DP), jnp.float32),
        compiler_params=pltpu.CompilerParams(vmem_limit_bytes=120 << 20),
    )(table_t)


def _mf_loss_body(u_hbm, i_hbm, r_hbm, w_hbm, h_hbm, out_hbm,
                  u_v, i_v, rv, wrows, hrows, acc_v, sem0, sem1):
    c = lax.axis_index("c")
    s = lax.axis_index("s")
    wid = s * NC + c
    sems = (sem0, sem1)

    pltpu.sync_copy(u_hbm.at[wid], u_v)
    pltpu.sync_copy(i_hbm.at[wid], i_v)
    pltpu.sync_copy(r_hbm.at[wid], rv)

    def fetch(chunk, b):
        pltpu.async_copy(w_hbm.at[u_v.at[chunk]], wrows.at[b], sems[b])
        pltpu.async_copy(h_hbm.at[i_v.at[chunk]], hrows.at[b], sems[b])

    def wait_chunk(b):
        pltpu.make_async_copy(w_hbm.at[u_v.at[0]], wrows.at[b], sems[b]).wait()
        pltpu.make_async_copy(h_hbm.at[i_v.at[0]], hrows.at[b], sems[b]).wait()

    lane = lax.iota(jnp.int32, L)
    row0 = wid * BPW

    fetch(0, 0)
    fetch(1, 1)
    lacc = jnp.zeros((L,), jnp.float32)
    for chunk in range(NCHK):
        b = chunk % 2
        wait_chunk(b)
        wch = wrows.at[b]
        hch = hrows.at[b]

        def group_body(g, acc, _wch=wch, _hch=hch, _chunk=chunk):
            rows = g * L + lane
            dot = jnp.zeros((L,), jnp.float32)
            for d in range(D):
                dvec = jnp.full((L,), d, jnp.int32)
                wv = plsc.load_gather(_wch, [rows, dvec])
                hv = plsc.load_gather(_hch, [rows, dvec])
                dot = dot + wv * hv
            rvals = rv[pl.ds(_chunk * CH + g * L, L)]
            err = dot - rvals
            grow = row0 + _chunk * CH + g * L + lane
            sq = jnp.where(grow == 0, jnp.float32(0), err * err)
            return acc + sq

        lacc = lax.fori_loop(0, CGROUPS, group_body, lacc)
        if chunk + 2 < NCHK:
            fetch(chunk + 2, b)

    acc_v[...] = lacc
    pltpu.sync_copy(acc_v, out_hbm.at[wid])


def kernel(u, i, r, W, H, U):
    del U
    u32 = u.astype(jnp.int32).reshape(NW, NCHK, CH)
    i32 = i.astype(jnp.int32).reshape(NW, NCHK, CH)
    r2 = r.astype(jnp.float32).reshape(NW, BPW)
    w_rm = _to_row_major(W.T)
    h_rm = _to_row_major(H.T)
    mesh = plsc.VectorSubcoreMesh(core_axis_name="c", subcore_axis_name="s")
    partials = pl.kernel(
        _mf_loss_body,
        out_type=jax.ShapeDtypeStruct((NW, L), jnp.float32),
        mesh=mesh,
        compiler_params=pltpu.CompilerParams(needs_layout_passes=False),
        scratch_types=[
            pltpu.VMEM((NCHK, CH), jnp.int32),
            pltpu.VMEM((NCHK, CH), jnp.int32),
            pltpu.VMEM((BPW,), jnp.float32),
            pltpu.VMEM((2, CH, DP), jnp.float32),
            pltpu.VMEM((2, CH, DP), jnp.float32),
            pltpu.VMEM((L,), jnp.float32),
            pltpu.SemaphoreType.DMA,
            pltpu.SemaphoreType.DMA,
        ],
    )(u32, i32, r2, w_rm, h_rm)
    return jnp.sum(partials) / BATCH
